# baseline (speedup 1.0000x reference)
import functools

import jax
import jax.numpy as jnp
from jax import lax
from jax.experimental import pallas as pl
from jax.experimental.pallas import tpu as pltpu
from jax.experimental.pallas import tpu_sc as plsc

D = 768
MAXPOS = 1025
B = 64
L = 1024
N = B * L
EPS = 1e-12

NC, NS, LANES = 2, 16, 16
NW = NC * NS
TPW = N // NW
K = 16
NCHUNK = TPW // K
NV = D // LANES
INV_D = 1.0 / D


def _comb_body(pos_ref, type_ref, main_ref, pad_ref):
    main_ref[...] = pos_ref[pl.ds(1, L), :] + type_ref[1:2, :]
    pad_ref[...] = pos_ref[0:1, :] + type_ref[0:1, :]


_comb_call = pl.pallas_call(
    _comb_body,
    out_shape=(
        jax.ShapeDtypeStruct((L, D), jnp.float32),
        jax.ShapeDtypeStruct((1, D), jnp.float32),
    ),
)


def _rsqrt16(a):
    yi = plsc.bitcast(a, jnp.int32)
    magic = jnp.full((LANES,), 0x5F3759DF, dtype=jnp.int32)
    y = plsc.bitcast(magic - lax.shift_right_logical(yi, 1), jnp.float32)
    half = a * 0.5
    for _ in range(3):
        y = y * (1.5 - half * y * y)
    return y


def _splat_i32(x):
    return jnp.full((LANES,), x, dtype=jnp.int32)


_mesh = plsc.VectorSubcoreMesh(core_axis_name="c", subcore_axis_name="s")


@functools.partial(
    pl.kernel,
    mesh=_mesh,
    compiler_params=pltpu.CompilerParams(needs_layout_passes=False),
    out_type=jax.ShapeDtypeStruct((N, D), jnp.float32),
    scratch_types=[
        pltpu.VMEM((2, K), jnp.int32),
        pltpu.VMEM((2, K, D), jnp.float32),
        pltpu.VMEM((2, K, D), jnp.float32),
        pltpu.VMEM((1, D), jnp.float32),
        pltpu.VMEM((K, LANES), jnp.float32),
        pltpu.VMEM((K, LANES), jnp.float32),
        pltpu.VMEM((K,), jnp.float32),
        pltpu.VMEM((K,), jnp.float32),
        pltpu.VMEM((K,), jnp.float32),
        pltpu.VMEM((D,), jnp.float32),
        pltpu.VMEM((D,), jnp.float32),
        pltpu.VMEM_SHARED((L, D), jnp.float32),
        pltpu.SemaphoreType.DMA,
        pltpu.SemaphoreType.DMA,
        pltpu.SemaphoreType.DMA,
        pltpu.SemaphoreType.DMA,
        pltpu.SemaphoreType.DMA,
        pltpu.SemaphoreType.DMA,
    ],
)
def _sc_embed(ids_hbm, tok_hbm, comb_hbm, pad_hbm, gamma_hbm, beta_hbm, out_hbm,
              ids_v, tok_v, comb_v, row0_v, sb_v, ssb_v, mk_v, mean_v, inv_v,
              gam_v, bet_v, comb_sh, st0, st1, sc0, sc1, so0, so1):
    wid = lax.axis_index("s") * NC + lax.axis_index("c")
    base = wid * TPW
    sem_tok = (st0, st1)
    sem_comb = (sc0, sc1)
    sem_out = (so0, so1)
    pltpu.sync_copy(gamma_hbm, gam_v)
    pltpu.sync_copy(beta_hbm, bet_v)

    @pl.when(lax.axis_index("s") == 0)
    def _load_comb():
        pltpu.sync_copy(comb_hbm, comb_sh)

    plsc.subcore_barrier()
    pltpu.sync_copy(pad_hbm, row0_v)

    zero = jnp.zeros((LANES,), jnp.float32)

    def fire_chunk(c, p):
        cbase = base + c * K
        posbase = lax.rem(cbase, L)
        pltpu.sync_copy(ids_hbm.at[pl.ds(cbase, K)], ids_v.at[p])
        pltpu.async_copy(tok_hbm.at[ids_v.at[p]], tok_v.at[p], sem_tok[p])
        pltpu.async_copy(comb_sh.at[pl.ds(posbase, K)], comb_v.at[p],
                         sem_comb[p])

    def wait_gathers(p):
        pltpu.make_async_copy(tok_hbm.at[pl.ds(0, K)], tok_v.at[p], sem_tok[p]).wait()
        pltpu.make_async_copy(tok_hbm.at[pl.ds(0, K)], comb_v.at[p], sem_comb[p]).wait()

    def wait_out(p):
        pltpu.make_async_copy(tok_hbm.at[pl.ds(0, K)], tok_v.at[p], sem_out[p]).wait()

    def compute_chunk(p):
        iv_ref = ids_v.at[p]
        tv = tok_v.at[p]
        cv = comb_v.at[p]

        idv16 = iv_ref[...]
        is_pad = idv16 == 0
        mk_v[...] = jnp.where(is_pad, 1.0, 0.0)
        npad = plsc.all_reduce_population_count(is_pad)[0]

        @pl.when(npad != 0)
        def _fix_pads():
            def fix(t, carry):
                mt = plsc.load_gather(mk_v, [_splat_i32(t)]) != 0.0
                for v in range(NV):
                    r0 = row0_v[0, pl.ds(v * LANES, LANES)]
                    cur = cv[t, pl.ds(v * LANES, LANES)]
                    cv[t, pl.ds(v * LANES, LANES)] = jnp.where(mt, r0, cur)
                return carry

            lax.fori_loop(0, K, fix, 0)

        def pass1(t, carry):
            s = zero
            ss = zero
            for v in range(NV):
                x = tv[t, pl.ds(v * LANES, LANES)] + cv[t, pl.ds(v * LANES, LANES)]
                tv[t, pl.ds(v * LANES, LANES)] = x
                s = s + x
                ss = ss + x * x
            sb_v[t, :] = s
            ssb_v[t, :] = ss
            return carry

        lax.fori_loop(0, K, pass1, 0)

        for g in range(K // LANES):
            rows = g * LANES + lax.iota(jnp.int32, LANES)
            s_tot = zero
            ss_tot = zero
            for j in range(LANES):
                col = _splat_i32(j)
                s_tot = s_tot + plsc.load_gather(sb_v, [rows, col])
                ss_tot = ss_tot + plsc.load_gather(ssb_v, [rows, col])
            mean = s_tot * INV_D
            var = ss_tot * INV_D - mean * mean
            mean_v[pl.ds(g * LANES, LANES)] = mean
            inv_v[pl.ds(g * LANES, LANES)] = _rsqrt16(var + EPS)

        NBLK = 4
        VB = NV // NBLK
        for blk in range(NBLK):
            gs = [gam_v[pl.ds((blk * VB + v) * LANES, LANES)] for v in range(VB)]
            bs = [bet_v[pl.ds((blk * VB + v) * LANES, LANES)] for v in range(VB)]

            def pass2(t, carry, _gs=gs, _bs=bs, _blk=blk):
                mv = plsc.load_gather(mean_v, [_splat_i32(t)])
                iv = plsc.load_gather(inv_v, [_splat_i32(t)])
                for v in range(VB):
                    off = (_blk * VB + v) * LANES
                    x = tv[t, pl.ds(off, LANES)]
                    tv[t, pl.ds(off, LANES)] = (x - mv) * iv * _gs[v] + _bs[v]
                return carry

            lax.fori_loop(0, K, pass2, 0)

    fire_chunk(0, 0)

    def outer(cc, carry):
        for p in (0, 1):
            c = cc * 2 + p
            @pl.when(c + 1 < NCHUNK)
            def _prefetch():
                @pl.when(c >= 1)
                def _drain_out():
                    wait_out(1 - p)

                fire_chunk(c + 1, 1 - p)

            wait_gathers(p)
            compute_chunk(p)
            cbase = base + c * K
            pltpu.async_copy(tok_v.at[p], out_hbm.at[pl.ds(cbase, K)], sem_out[p])
        return carry

    lax.fori_loop(0, NCHUNK // 2, outer, 0)
    wait_out(0)
    wait_out(1)


def kernel(input_ids, tok_emb, pos_emb, type_emb, gamma, beta):
    comb, pad_row = _comb_call(pos_emb, type_emb)
    ids = input_ids.reshape(-1).astype(jnp.int32)
    out = _sc_embed(ids, tok_emb, comb, pad_row, gamma, beta)
    return out.reshape(input_ids.shape[0], input_ids.shape[1], D)


# speedup vs baseline: 1.1386x; 1.1386x over previous
import functools

import jax
import jax.numpy as jnp
from jax import lax
from jax.experimental import pallas as pl
from jax.experimental.pallas import tpu as pltpu
from jax.experimental.pallas import tpu_sc as plsc

D = 768
MAXPOS = 1025
B = 64
L = 1024
N = B * L
EPS = 1e-12

NC, NS, LANES = 2, 16, 16
NW = NC * NS
TPW = N // NW
K = 32
NCHUNK = TPW // K
NV = D // LANES
INV_D = 1.0 / D


def _comb_body(pos_ref, type_ref, main_ref, pad_ref):
    main_ref[...] = pos_ref[pl.ds(1, L), :] + type_ref[1:2, :]
    pad_ref[...] = pos_ref[0:1, :] + type_ref[0:1, :]


_comb_call = pl.pallas_call(
    _comb_body,
    out_shape=(
        jax.ShapeDtypeStruct((L, D), jnp.float32),
        jax.ShapeDtypeStruct((1, D), jnp.float32),
    ),
)


def _rsqrt16(a):
    yi = plsc.bitcast(a, jnp.int32)
    magic = jnp.full((LANES,), 0x5F3759DF, dtype=jnp.int32)
    y = plsc.bitcast(magic - lax.shift_right_logical(yi, 1), jnp.float32)
    half = a * 0.5
    for _ in range(3):
        y = y * (1.5 - half * y * y)
    return y


def _splat_i32(x):
    return jnp.full((LANES,), x, dtype=jnp.int32)


_mesh = plsc.VectorSubcoreMesh(core_axis_name="c", subcore_axis_name="s")


@functools.partial(
    pl.kernel,
    mesh=_mesh,
    compiler_params=pltpu.CompilerParams(needs_layout_passes=False),
    out_type=jax.ShapeDtypeStruct((N, D), jnp.float32),
    scratch_types=[
        pltpu.VMEM((2, K), jnp.int32),
        pltpu.VMEM((2, K, D), jnp.float32),
        pltpu.VMEM((2, K, D), jnp.float32),
        pltpu.VMEM((1, D), jnp.float32),
        pltpu.VMEM((K, LANES), jnp.float32),
        pltpu.VMEM((K, LANES), jnp.float32),
        pltpu.VMEM((K,), jnp.float32),
        pltpu.VMEM((K,), jnp.float32),
        pltpu.VMEM((K,), jnp.float32),
        pltpu.VMEM((D,), jnp.float32),
        pltpu.VMEM((D,), jnp.float32),
        pltpu.SemaphoreType.DMA,
        pltpu.SemaphoreType.DMA,
        pltpu.SemaphoreType.DMA,
        pltpu.SemaphoreType.DMA,
        pltpu.SemaphoreType.DMA,
        pltpu.SemaphoreType.DMA,
    ],
)
def _sc_embed(ids_hbm, tok_hbm, comb_hbm, pad_hbm, gamma_hbm, beta_hbm, out_hbm,
              ids_v, tok_v, comb_v, row0_v, sb_v, ssb_v, mk_v, mean_v, inv_v,
              gam_v, bet_v, st0, st1, sc0, sc1, so0, so1):
    wid = lax.axis_index("s") * NC + lax.axis_index("c")
    base = wid * TPW
    sem_tok = (st0, st1)
    sem_comb = (sc0, sc1)
    sem_out = (so0, so1)
    pltpu.sync_copy(gamma_hbm, gam_v)
    pltpu.sync_copy(beta_hbm, bet_v)

    pltpu.sync_copy(pad_hbm, row0_v)

    zero = jnp.zeros((LANES,), jnp.float32)

    def fire_chunk(c, p):
        cbase = base + c * K
        posbase = lax.rem(cbase, L)
        pltpu.sync_copy(ids_hbm.at[pl.ds(cbase, K)], ids_v.at[p])
        pltpu.async_copy(tok_hbm.at[ids_v.at[p]], tok_v.at[p], sem_tok[p])
        pltpu.async_copy(comb_hbm.at[pl.ds(posbase, K)], comb_v.at[p],
                         sem_comb[p])

    def wait_gathers(p):
        pltpu.make_async_copy(tok_hbm.at[pl.ds(0, K)], tok_v.at[p], sem_tok[p]).wait()
        pltpu.make_async_copy(tok_hbm.at[pl.ds(0, K)], comb_v.at[p], sem_comb[p]).wait()

    def wait_out(p):
        pltpu.make_async_copy(tok_hbm.at[pl.ds(0, K)], tok_v.at[p], sem_out[p]).wait()

    def compute_chunk(p):
        iv_ref = ids_v.at[p]
        tv = tok_v.at[p]
        cv = comb_v.at[p]

        npad = jnp.int32(0)
        for g in range(K // LANES):
            idv = iv_ref[pl.ds(g * LANES, LANES)]
            is_pad = idv == 0
            mk_v[pl.ds(g * LANES, LANES)] = jnp.where(is_pad, 1.0, 0.0)
            npad = npad + plsc.all_reduce_population_count(is_pad)[0]

        @pl.when(npad != 0)
        def _fix_pads():
            def fix(t, carry):
                mt = plsc.load_gather(mk_v, [_splat_i32(t)]) != 0.0
                for v in range(NV):
                    r0 = row0_v[0, pl.ds(v * LANES, LANES)]
                    cur = cv[t, pl.ds(v * LANES, LANES)]
                    cv[t, pl.ds(v * LANES, LANES)] = jnp.where(mt, r0, cur)
                return carry

            lax.fori_loop(0, K, fix, 0)

        def pass1(t, carry):
            s = zero
            ss = zero
            for v in range(NV):
                x = tv[t, pl.ds(v * LANES, LANES)] + cv[t, pl.ds(v * LANES, LANES)]
                tv[t, pl.ds(v * LANES, LANES)] = x
                s = s + x
                ss = ss + x * x
            sb_v[t, :] = s
            ssb_v[t, :] = ss
            return carry

        lax.fori_loop(0, K, pass1, 0)

        for g in range(K // LANES):
            rows = g * LANES + lax.iota(jnp.int32, LANES)
            s_tot = zero
            ss_tot = zero
            for j in range(LANES):
                col = _splat_i32(j)
                s_tot = s_tot + plsc.load_gather(sb_v, [rows, col])
                ss_tot = ss_tot + plsc.load_gather(ssb_v, [rows, col])
            mean = s_tot * INV_D
            var = ss_tot * INV_D - mean * mean
            mean_v[pl.ds(g * LANES, LANES)] = mean
            inv_v[pl.ds(g * LANES, LANES)] = _rsqrt16(var + EPS)

        NBLK = 4
        VB = NV // NBLK
        for blk in range(NBLK):
            gs = [gam_v[pl.ds((blk * VB + v) * LANES, LANES)] for v in range(VB)]
            bs = [bet_v[pl.ds((blk * VB + v) * LANES, LANES)] for v in range(VB)]

            def pass2(t, carry, _gs=gs, _bs=bs, _blk=blk):
                mv = plsc.load_gather(mean_v, [_splat_i32(t)])
                iv = plsc.load_gather(inv_v, [_splat_i32(t)])
                for v in range(VB):
                    off = (_blk * VB + v) * LANES
                    x = tv[t, pl.ds(off, LANES)]
                    tv[t, pl.ds(off, LANES)] = (x - mv) * iv * _gs[v] + _bs[v]
                return carry

            lax.fori_loop(0, K, pass2, 0)

    fire_chunk(0, 0)

    def outer(cc, carry):
        for p in (0, 1):
            c = cc * 2 + p
            @pl.when(c + 1 < NCHUNK)
            def _prefetch():
                @pl.when(c >= 1)
                def _drain_out():
                    wait_out(1 - p)

                fire_chunk(c + 1, 1 - p)

            wait_gathers(p)
            compute_chunk(p)
            cbase = base + c * K
            pltpu.async_copy(tok_v.at[p], out_hbm.at[pl.ds(cbase, K)], sem_out[p])
        return carry

    lax.fori_loop(0, NCHUNK // 2, outer, 0)
    wait_out(0)
    wait_out(1)


def kernel(input_ids, tok_emb, pos_emb, type_emb, gamma, beta):
    comb, pad_row = _comb_call(pos_emb, type_emb)
    ids = input_ids.reshape(-1).astype(jnp.int32)
    out = _sc_embed(ids, tok_emb, comb, pad_row, gamma, beta)
    return out.reshape(input_ids.shape[0], input_ids.shape[1], D)
